# Initial kernel scaffold; baseline (speedup 1.0000x reference)
import jax
import jax.numpy as jnp
from jax.experimental import pallas as pl


def kernel(pillar_features, voxel_coords, batch_size, Wq, bq, Wk, bk, Wv, bv, W1, b1, W2, b2, g1, beta1, g2, beta2):
    raise NotImplementedError("write your pallas kernel here")


# speedup vs baseline: 1.9622x; 1.9622x over previous
import functools
import math

import jax
import jax.numpy as jnp
from jax import lax
from jax.experimental import pallas as pl
from jax.experimental.pallas import tpu as pltpu
from jax.experimental.pallas import tpu_sc as plsc

_NX, _NY = 216, 248
_C, _HID = 64, 64
_P = 4000
_B = 2
_NCELL = _NY * _NX
_QB = 400
_TB = 1792
_NW = 16
_PW = (_P * _B) // _NW
_MW = (_B * _NCELL) // _NW
_MWPAD = ((_MW + 15) // 16) * 16


def _prep_body(hh_ref, ww_ref, p_ref, g1_ref, be1_ref, wq_ref, bq_ref,
               wk_ref, bk_ref, wv_ref, bv_ref,
               ne1_ref, q_ref, k_ref, v_ref):
    half = _C // 2
    j = lax.broadcasted_iota(jnp.int32, (1, half), 1).astype(jnp.float32)
    theta = jnp.exp(j * (-math.log(10000.0) / half))
    ah = hh_ref[...] * theta
    aw = ww_ref[...] * theta
    hs, hc = jnp.sin(ah), jnp.cos(ah)
    ws, wc = jnp.sin(aw), jnp.cos(aw)
    cc = hc * wc
    ss = hs * ws
    p = p_ref[...]
    x1 = p[:, :half]
    x2 = p[:, half:]
    ne1 = jnp.concatenate([x1 + x1 * cc - x2 * ss,
                           x2 + x1 * ss + x2 * cc], axis=1)
    mu = jnp.mean(ne1, axis=-1, keepdims=True)
    d = ne1 - mu
    var = jnp.mean(d * d, axis=-1, keepdims=True)
    ne = d * lax.rsqrt(var + 1e-5) * g1_ref[...] + be1_ref[...]
    ne1_ref[...] = ne1
    q_ref[...] = jnp.dot(ne, wq_ref[...], preferred_element_type=jnp.float32) + bq_ref[...]
    k_ref[...] = jnp.dot(ne, wk_ref[...], preferred_element_type=jnp.float32) + bk_ref[...]
    v_ref[...] = jnp.dot(ne, wv_ref[...], preferred_element_type=jnp.float32) + bv_ref[...]


def _att_body(q_ref, ne1_ref, k_ref, v_ref, g2_ref, be2_ref,
              w1_ref, b1_ref, w2_ref, b2_ref, up_ref):
    s = lax.dot_general(q_ref[...], k_ref[...], (((1,), (1,)), ((), ())),
                        preferred_element_type=jnp.float32)
    m = jnp.max(s, axis=-1, keepdims=True)
    e = jnp.exp(s - m)
    l = jnp.sum(e, axis=-1, keepdims=True)
    o = jnp.dot(e, v_ref[...], preferred_element_type=jnp.float32) / l
    att1 = o + ne1_ref[...]
    mu = jnp.mean(att1, axis=-1, keepdims=True)
    d = att1 - mu
    var = jnp.mean(d * d, axis=-1, keepdims=True)
    t = d * lax.rsqrt(var + 1e-5) * g2_ref[...] + be2_ref[...]
    h = jnp.dot(t, w1_ref[...], preferred_element_type=jnp.float32) + b1_ref[...]
    h = h * 0.5 * (1.0 + lax.erf(h * (1.0 / math.sqrt(2.0))))
    up_ref[...] = (jnp.dot(h, w2_ref[...], preferred_element_type=jnp.float32)
                   + b2_ref[...] + att1)


def _sc_scatter_body(up_hbm, idxg_hbm, dense_hbm, mask_hbm,
                     zero_v, idx_v, rows_v, ones_v, sem):
    w = lax.axis_index("s")

    zvec = jnp.zeros((16,), jnp.float32)
    def _zb(i, c):
        zero_v[pl.ds(i * 16, 16)] = zvec
        return c
    lax.fori_loop(0, _MWPAD // 16, _zb, 0)
    ovec = jnp.ones((16,), jnp.float32)
    def _ob(i, c):
        ones_v[pl.ds(i * 16, 16)] = ovec
        return c
    lax.fori_loop(0, ((_PW + 15) // 16 * 16) // 16, _ob, 0)

    pltpu.sync_copy(zero_v.at[pl.ds(0, _MW)], mask_hbm.at[pl.ds(w * _MW, _MW)])
    plsc.subcore_barrier()

    pltpu.sync_copy(idxg_hbm.at[w], idx_v)
    pltpu.sync_copy(up_hbm.at[w], rows_v)
    pltpu.async_copy(rows_v, dense_hbm.at[idx_v], sem).wait()
    pltpu.async_copy(ones_v.at[pl.ds(0, _PW)], mask_hbm.at[idx_v], sem).wait()


def _sel_t_body(dense_ref, mask_ref, out_ref):
    vals = dense_ref[...]
    m = mask_ref[...]
    out_ref[...] = jnp.where(m > 0.5, vals.T, 0.0)


def kernel(pillar_features, voxel_coords, batch_size, Wq, bq, Wk, bk, Wv, bv,
           W1, b1, W2, b2, g1, beta1, g2, beta2):
    f32 = jnp.float32
    p = pillar_features.astype(f32).reshape(_B, _P, _C)
    vc = voxel_coords.astype(jnp.int32)
    y = vc[:, 2].astype(f32).reshape(_B, _P, 1)
    x = vc[:, 3].astype(f32).reshape(_B, _P, 1)
    hh = y * (2.0 / (_NY - 1)) - 1.0
    ww = x * (2.0 / (_NX - 1)) - 1.0
    idxg = (vc[:, 0] * _NCELL + vc[:, 1] + vc[:, 2] * _NX + vc[:, 3]).reshape(_NW, _PW)

    wspec = lambda *shape: pl.BlockSpec(shape, lambda *a: (0,) * len(shape))
    bspec = pl.BlockSpec((None, _P, _C), lambda b: (b, 0, 0))
    hspec = pl.BlockSpec((None, _P, 1), lambda b: (b, 0, 0))
    ne1, q, k, v = pl.pallas_call(
        _prep_body,
        grid=(_B,),
        in_specs=[hspec, hspec, bspec,
                  wspec(_C), wspec(_C),
                  wspec(_C, _HID), wspec(_HID),
                  wspec(_C, _HID), wspec(_HID),
                  wspec(_C, _HID), wspec(_HID)],
        out_specs=[bspec] * 4,
        out_shape=[jax.ShapeDtypeStruct((_B, _P, _C), f32)] * 4,
    )(hh, ww, p, g1, beta1, Wq, bq, Wk, bk, Wv, bv)

    nq = _P // _QB
    wspec2 = lambda *shape: pl.BlockSpec(shape, lambda *a: (0,) * len(shape))
    qspec = pl.BlockSpec((None, _QB, _C), lambda b, i: (b, i, 0))
    kspec = pl.BlockSpec((None, _P, _C), lambda b, i: (b, 0, 0))
    up = pl.pallas_call(
        _att_body,
        grid=(_B, nq),
        in_specs=[qspec, qspec, kspec, kspec,
                  wspec2(_C), wspec2(_C),
                  wspec2(_HID, _HID), wspec2(_HID),
                  wspec2(_HID, _C), wspec2(_C)],
        out_specs=qspec,
        out_shape=jax.ShapeDtypeStruct((_B, _P, _C), f32),
    )(q, ne1, k, v, g2, beta2, W1, b1, W2, b2)

    mesh = plsc.VectorSubcoreMesh(core_axis_name="c", subcore_axis_name="s",
                                  num_cores=1)
    dense, mask = pl.kernel(
        _sc_scatter_body,
        out_type=(jax.ShapeDtypeStruct((_B * _NCELL, _C), f32),
                  jax.ShapeDtypeStruct((_B * _NCELL,), f32)),
        mesh=mesh,
        scratch_types=[pltpu.VMEM((_MWPAD,), f32),
                       pltpu.VMEM((_PW,), jnp.int32),
                       pltpu.VMEM((_PW, _C), f32),
                       pltpu.VMEM(((_PW + 15) // 16 * 16,), f32),
                       pltpu.SemaphoreType.DMA],
        compiler_params=pltpu.CompilerParams(use_tc_tiling_on_sc=False),
    )(up.reshape(_NW, _PW, _C), idxg)

    nt = (_NCELL + _TB - 1) // _TB
    out = pl.pallas_call(
        _sel_t_body,
        grid=(_B, nt),
        in_specs=[pl.BlockSpec((None, _TB, _C), lambda b, j: (b, j, 0)),
                  pl.BlockSpec((None, 1, _TB), lambda b, j: (b, 0, j))],
        out_specs=pl.BlockSpec((None, _C, _TB), lambda b, j: (b, 0, j)),
        out_shape=jax.ShapeDtypeStruct((_B, _C, _NCELL), f32),
    )(dense.reshape(_B, _NCELL, _C), mask.reshape(_B, 1, _NCELL))
    return out.reshape(_B, _C, _NY, _NX)


# speedup vs baseline: 2.2484x; 1.1458x over previous
import functools
import math

import jax
import jax.numpy as jnp
from jax import lax
from jax.experimental import pallas as pl
from jax.experimental.pallas import tpu as pltpu
from jax.experimental.pallas import tpu_sc as plsc

_NX, _NY = 216, 248
_C, _HID = 64, 64
_CP = 128
_P = 4000
_B = 2
_NCELL = _NY * _NX
_QB = 400
_TB = 2048
_NTB = 27
_NCELLP = _TB * _NTB
_NWA = 16
_PWA = (_P * _B) // _NWA
_MW = (_B * _NCELLP) // _NWA
_NWB = 32
_PWB = (_P * _B) // _NWB


def _prep_body(hh_ref, ww_ref, p_ref, g1_ref, be1_ref, wq_ref, bq_ref,
               wk_ref, bk_ref, wv_ref, bv_ref,
               ne1_ref, q_ref, k_ref, v_ref):
    half = _C // 2
    j = lax.broadcasted_iota(jnp.int32, (1, half), 1).astype(jnp.float32)
    theta = jnp.exp(j * (-math.log(10000.0) / half))
    ah = hh_ref[...] * theta
    aw = ww_ref[...] * theta
    hs, hc = jnp.sin(ah), jnp.cos(ah)
    ws, wc = jnp.sin(aw), jnp.cos(aw)
    cc = hc * wc
    ss = hs * ws
    p = p_ref[...]
    x1 = p[:, :half]
    x2 = p[:, half:]
    ne1 = jnp.concatenate([x1 + x1 * cc - x2 * ss,
                           x2 + x1 * ss + x2 * cc], axis=1)
    mu = jnp.mean(ne1, axis=-1, keepdims=True)
    d = ne1 - mu
    var = jnp.mean(d * d, axis=-1, keepdims=True)
    ne = d * lax.rsqrt(var + 1e-5) * g1_ref[...] + be1_ref[...]
    ne1_ref[...] = ne1
    q_ref[...] = jnp.dot(ne, wq_ref[...], preferred_element_type=jnp.float32) + bq_ref[...]
    k_ref[...] = jnp.dot(ne, wk_ref[...], preferred_element_type=jnp.float32) + bk_ref[...]
    v_ref[...] = jnp.dot(ne, wv_ref[...], preferred_element_type=jnp.float32) + bv_ref[...]


def _att_body(q_ref, ne1_ref, k_ref, v_ref, g2_ref, be2_ref,
              w1_ref, b1_ref, w2_ref, b2_ref, up_ref):
    s = lax.dot_general(q_ref[...], k_ref[...], (((1,), (1,)), ((), ())),
                        preferred_element_type=jnp.float32)
    m = jnp.max(s, axis=-1, keepdims=True)
    e = jnp.exp(s - m)
    l = jnp.sum(e, axis=-1, keepdims=True)
    o = jnp.dot(e, v_ref[...], preferred_element_type=jnp.float32) / l
    att1 = o + ne1_ref[...]
    mu = jnp.mean(att1, axis=-1, keepdims=True)
    d = att1 - mu
    var = jnp.mean(d * d, axis=-1, keepdims=True)
    t = d * lax.rsqrt(var + 1e-5) * g2_ref[...] + be2_ref[...]
    h = jnp.dot(t, w1_ref[...], preferred_element_type=jnp.float32) + b1_ref[...]
    h = h * 0.5 * (1.0 + lax.erf(h * (1.0 / math.sqrt(2.0))))
    up = (jnp.dot(h, w2_ref[...], preferred_element_type=jnp.float32)
          + b2_ref[...] + att1)
    up_ref[...] = jnp.concatenate([up, jnp.zeros_like(up)], axis=1)


def _sc_mask_body(idxg_hbm, mask_hbm, zero_v, idx_v, ones_v, sem):
    w = lax.axis_index("s")

    zvec = jnp.zeros((16,), jnp.float32)
    def _zb(i, c):
        zero_v[pl.ds(i * 16, 16)] = zvec
        return c
    lax.fori_loop(0, _MW // 16, _zb, 0)
    ovec = jnp.ones((16,), jnp.float32)
    def _ob(i, c):
        ones_v[pl.ds(i * 16, 16)] = ovec
        return c
    lax.fori_loop(0, ((_PWA + 15) // 16 * 16) // 16, _ob, 0)

    pltpu.sync_copy(zero_v, mask_hbm.at[pl.ds(w * _MW, _MW)])
    plsc.subcore_barrier()

    pltpu.sync_copy(idxg_hbm.at[w], idx_v)
    pltpu.async_copy(ones_v.at[pl.ds(0, _PWA)], mask_hbm.at[idx_v], sem).wait()


def _sc_scatter_body(up_hbm, idxg_hbm, dense_hbm, idx_v, rows_v, sem):
    cid = lax.axis_index("c")
    sid = lax.axis_index("s")
    w = sid * 2 + cid
    pltpu.sync_copy(idxg_hbm.at[w], idx_v)
    pltpu.sync_copy(up_hbm.at[w], rows_v)
    pltpu.async_copy(rows_v, dense_hbm.at[idx_v], sem).wait()


def _sel_t_body(dense_ref, mask_ref, out_ref):
    vals = dense_ref[...]
    m = mask_ref[...]
    out_ref[...] = jnp.where(m[None, :] > 0.5, vals[:, :_C].T, 0.0)


def kernel(pillar_features, voxel_coords, batch_size, Wq, bq, Wk, bk, Wv, bv,
           W1, b1, W2, b2, g1, beta1, g2, beta2):
    f32 = jnp.float32
    p = pillar_features.astype(f32).reshape(_B, _P, _C)
    vc = voxel_coords.astype(jnp.int32)
    y = vc[:, 2].astype(f32).reshape(_B, _P, 1)
    x = vc[:, 3].astype(f32).reshape(_B, _P, 1)
    hh = y * (2.0 / (_NY - 1)) - 1.0
    ww = x * (2.0 / (_NX - 1)) - 1.0
    idxg = (vc[:, 0] * _NCELLP + vc[:, 1] + vc[:, 2] * _NX + vc[:, 3])

    wspec = lambda *shape: pl.BlockSpec(shape, lambda *a: (0,) * len(shape))
    bspec = pl.BlockSpec((None, _P, _C), lambda b: (b, 0, 0))
    hspec = pl.BlockSpec((None, _P, 1), lambda b: (b, 0, 0))
    ne1, q, k, v = pl.pallas_call(
        _prep_body,
        grid=(_B,),
        in_specs=[hspec, hspec, bspec,
                  wspec(_C), wspec(_C),
                  wspec(_C, _HID), wspec(_HID),
                  wspec(_C, _HID), wspec(_HID),
                  wspec(_C, _HID), wspec(_HID)],
        out_specs=[bspec] * 4,
        out_shape=[jax.ShapeDtypeStruct((_B, _P, _C), f32)] * 4,
    )(hh, ww, p, g1, beta1, Wq, bq, Wk, bk, Wv, bv)

    nq = _P // _QB
    qspec = pl.BlockSpec((None, _QB, _C), lambda b, i: (b, i, 0))
    kspec = pl.BlockSpec((None, _P, _C), lambda b, i: (b, 0, 0))
    up = pl.pallas_call(
        _att_body,
        grid=(_B, nq),
        in_specs=[qspec, qspec, kspec, kspec,
                  wspec(_C), wspec(_C),
                  wspec(_HID, _HID), wspec(_HID),
                  wspec(_HID, _C), wspec(_C)],
        out_specs=pl.BlockSpec((None, _QB, _CP), lambda b, i: (b, i, 0)),
        out_shape=jax.ShapeDtypeStruct((_B, _P, _CP), f32),
    )(q, ne1, k, v, g2, beta2, W1, b1, W2, b2)

    mesh1 = plsc.VectorSubcoreMesh(core_axis_name="c", subcore_axis_name="s",
                                   num_cores=1)
    mask = pl.kernel(
        _sc_mask_body,
        out_type=jax.ShapeDtypeStruct((_B * _NCELLP,), f32),
        mesh=mesh1,
        scratch_types=[pltpu.VMEM((_MW,), f32),
                       pltpu.VMEM((_PWA,), jnp.int32),
                       pltpu.VMEM(((_PWA + 15) // 16 * 16,), f32),
                       pltpu.SemaphoreType.DMA],
        compiler_params=pltpu.CompilerParams(use_tc_tiling_on_sc=False),
    )(idxg.reshape(_NWA, _PWA))

    mesh2 = plsc.VectorSubcoreMesh(core_axis_name="c", subcore_axis_name="s")
    dense = pl.kernel(
        _sc_scatter_body,
        out_type=jax.ShapeDtypeStruct((_B * _NCELLP, _CP), f32),
        mesh=mesh2,
        scratch_types=[pltpu.VMEM((_PWB,), jnp.int32),
                       pltpu.VMEM((_PWB, _CP), f32),
                       pltpu.SemaphoreType.DMA],
    )(up.reshape(_NWB, _PWB, _CP), idxg.reshape(_NWB, _PWB))

    out = pl.pallas_call(
        _sel_t_body,
        grid=(_B, _NTB),
        in_specs=[pl.BlockSpec((_TB, _CP), lambda b, j: (b * _NTB + j, 0)),
                  pl.BlockSpec((_TB,), lambda b, j: (b * _NTB + j,))],
        out_specs=pl.BlockSpec((None, _C, _TB), lambda b, j: (b, 0, j)),
        out_shape=jax.ShapeDtypeStruct((_B, _C, _NCELL), f32),
    )(dense, mask)
    return out.reshape(_B, _C, _NY, _NX)


# speedup vs baseline: 3.7835x; 1.6827x over previous
import functools
import math

import jax
import jax.numpy as jnp
from jax import lax
from jax.experimental import pallas as pl
from jax.experimental.pallas import tpu as pltpu
from jax.experimental.pallas import tpu_sc as plsc

_NX, _NY = 216, 248
_C, _HID = 64, 64
_CP = 128
_P = 4000
_B = 2
_NCELL = _NY * _NX
_QB = 400
_YB = 8
_NYB = _NY // _YB
_TB = _YB * _NX
_NWA = 16
_PWA = (_P * _B) // _NWA
_MW = (_B * _NCELL) // _NWA
_MWPAD = ((_MW + 15) // 16) * 16
_NWB = 20
_PWB = (_P * _B) // _NWB


def _prep_body(hh_ref, ww_ref, p_ref, g1_ref, be1_ref, wq_ref, bq_ref,
               wk_ref, bk_ref, wv_ref, bv_ref,
               ne1_ref, q_ref, k_ref, v_ref):
    half = _C // 2
    j = lax.broadcasted_iota(jnp.int32, (1, half), 1).astype(jnp.float32)
    theta = jnp.exp(j * (-math.log(10000.0) / half))
    ah = hh_ref[...] * theta
    aw = ww_ref[...] * theta
    hs, hc = jnp.sin(ah), jnp.cos(ah)
    ws, wc = jnp.sin(aw), jnp.cos(aw)
    cc = hc * wc
    ss = hs * ws
    p = p_ref[...]
    x1 = p[:, :half]
    x2 = p[:, half:]
    ne1 = jnp.concatenate([x1 + x1 * cc - x2 * ss,
                           x2 + x1 * ss + x2 * cc], axis=1)
    mu = jnp.mean(ne1, axis=-1, keepdims=True)
    d = ne1 - mu
    var = jnp.mean(d * d, axis=-1, keepdims=True)
    ne = d * lax.rsqrt(var + 1e-5) * g1_ref[...] + be1_ref[...]
    ne1_ref[...] = ne1
    q_ref[...] = jnp.dot(ne, wq_ref[...], preferred_element_type=jnp.float32) + bq_ref[...]
    k_ref[...] = jnp.dot(ne, wk_ref[...], preferred_element_type=jnp.float32) + bk_ref[...]
    v = jnp.dot(ne, wv_ref[...], preferred_element_type=jnp.float32) + bv_ref[...]
    v_ref[...] = jnp.concatenate(
        [v, jnp.ones((_P, 1), jnp.float32), jnp.zeros((_P, _CP - _C - 1), jnp.float32)],
        axis=1)


def _att_body(q_ref, ne1_ref, k_ref, v_ref, g2_ref, be2_ref,
              w1_ref, b1_ref, w2_ref, b2_ref, up_ref):
    s = lax.dot_general(q_ref[...], k_ref[...], (((1,), (1,)), ((), ())),
                        preferred_element_type=jnp.float32)
    e = jnp.exp(s)
    o = jnp.dot(e, v_ref[...], preferred_element_type=jnp.float32)
    att1 = o[:, :_C] / o[:, _C:_C + 1] + ne1_ref[...]
    mu = jnp.mean(att1, axis=-1, keepdims=True)
    d = att1 - mu
    var = jnp.mean(d * d, axis=-1, keepdims=True)
    t = d * lax.rsqrt(var + 1e-5) * g2_ref[...] + be2_ref[...]
    h = jnp.dot(t, w1_ref[...], preferred_element_type=jnp.float32) + b1_ref[...]
    h = h * 0.5 * (1.0 + lax.erf(h * (1.0 / math.sqrt(2.0))))
    up = (jnp.dot(h, w2_ref[...], preferred_element_type=jnp.float32)
          + b2_ref[...] + att1)
    up_ref[...] = jnp.concatenate([up, jnp.zeros_like(up)], axis=1)


def _sc_mask_body(idxg_hbm, mask_hbm, zero_v, idx_v, ones_v, sem):
    w = lax.axis_index("s")

    zvec = jnp.zeros((16,), jnp.float32)
    def _zb(i, c):
        zero_v[pl.ds(i * 16, 16)] = zvec
        return c
    lax.fori_loop(0, _MWPAD // 16, _zb, 0)
    ovec = jnp.ones((16,), jnp.float32)
    def _ob(i, c):
        ones_v[pl.ds(i * 16, 16)] = ovec
        return c
    lax.fori_loop(0, ((_PWA + 15) // 16 * 16) // 16, _ob, 0)

    pltpu.sync_copy(zero_v.at[pl.ds(0, _MW)], mask_hbm.at[pl.ds(w * _MW, _MW)])
    plsc.subcore_barrier()

    pltpu.sync_copy(idxg_hbm.at[w], idx_v)
    pltpu.async_copy(ones_v.at[pl.ds(0, _PWA)], mask_hbm.at[idx_v], sem).wait()


def _sc_scatter_body(up_hbm, idxg_hbm, dense_hbm, idx_v, rows_v, sem):
    cid = lax.axis_index("c")
    sid = lax.axis_index("s")
    w = sid * 2 + cid

    @pl.when(w < _NWB)
    def _():
        pltpu.sync_copy(idxg_hbm.at[w], idx_v)
        pltpu.sync_copy(up_hbm.at[w], rows_v)
        pltpu.async_copy(rows_v, dense_hbm.at[idx_v], sem).wait()


def _fin_body(dense_ref, mask_ref, out_ref):
    vals = dense_ref[...]
    m = mask_ref[...]
    pieces = [vals[r * _NX:(r + 1) * _NX, :_C].T[:, None, :] for r in range(_YB)]
    t = jnp.concatenate(pieces, axis=1)
    out_ref[...] = jnp.where(m[None, :, :] > 0.5, t, 0.0)


def kernel(pillar_features, voxel_coords, batch_size, Wq, bq, Wk, bk, Wv, bv,
           W1, b1, W2, b2, g1, beta1, g2, beta2):
    f32 = jnp.float32
    p = pillar_features.astype(f32).reshape(_B, _P, _C)
    vc = voxel_coords.astype(jnp.int32)
    y = vc[:, 2].astype(f32).reshape(_B, _P, 1)
    x = vc[:, 3].astype(f32).reshape(_B, _P, 1)
    hh = y * (2.0 / (_NY - 1)) - 1.0
    ww = x * (2.0 / (_NX - 1)) - 1.0
    idxg = (vc[:, 0] * _NCELL + vc[:, 1] + vc[:, 2] * _NX + vc[:, 3])

    wspec = lambda *shape: pl.BlockSpec(shape, lambda *a: (0,) * len(shape))
    bspec = pl.BlockSpec((None, _P, _C), lambda b: (b, 0, 0))
    hspec = pl.BlockSpec((None, _P, 1), lambda b: (b, 0, 0))
    ne1, q, k, v = pl.pallas_call(
        _prep_body,
        grid=(_B,),
        in_specs=[hspec, hspec, bspec,
                  wspec(_C), wspec(_C),
                  wspec(_C, _HID), wspec(_HID),
                  wspec(_C, _HID), wspec(_HID),
                  wspec(_C, _HID), wspec(_HID)],
        out_specs=[bspec, bspec, bspec,
                   pl.BlockSpec((None, _P, _CP), lambda b: (b, 0, 0))],
        out_shape=[jax.ShapeDtypeStruct((_B, _P, _C), f32)] * 3
        + [jax.ShapeDtypeStruct((_B, _P, _CP), f32)],
    )(hh, ww, p, g1, beta1, Wq, bq, Wk, bk, Wv, bv)

    nq = _P // _QB
    qspec = pl.BlockSpec((None, _QB, _C), lambda b, i: (b, i, 0))
    up = pl.pallas_call(
        _att_body,
        grid=(_B, nq),
        in_specs=[qspec, qspec,
                  pl.BlockSpec((None, _P, _C), lambda b, i: (b, 0, 0)),
                  pl.BlockSpec((None, _P, _CP), lambda b, i: (b, 0, 0)),
                  wspec(_C), wspec(_C),
                  wspec(_HID, _HID), wspec(_HID),
                  wspec(_HID, _C), wspec(_C)],
        out_specs=pl.BlockSpec((None, _QB, _CP), lambda b, i: (b, i, 0)),
        out_shape=jax.ShapeDtypeStruct((_B, _P, _CP), f32),
    )(q, ne1, k, v, g2, beta2, W1, b1, W2, b2)

    mesh1 = plsc.VectorSubcoreMesh(core_axis_name="c", subcore_axis_name="s",
                                   num_cores=1)
    mask = pl.kernel(
        _sc_mask_body,
        out_type=jax.ShapeDtypeStruct((_B * _NCELL,), f32),
        mesh=mesh1,
        scratch_types=[pltpu.VMEM((_MWPAD,), f32),
                       pltpu.VMEM((_PWA,), jnp.int32),
                       pltpu.VMEM(((_PWA + 15) // 16 * 16,), f32),
                       pltpu.SemaphoreType.DMA],
        compiler_params=pltpu.CompilerParams(use_tc_tiling_on_sc=False),
    )(idxg.reshape(_NWA, _PWA))

    mesh2 = plsc.VectorSubcoreMesh(core_axis_name="c", subcore_axis_name="s")
    dense = pl.kernel(
        _sc_scatter_body,
        out_type=jax.ShapeDtypeStruct((_B * _NCELL, _CP), f32),
        mesh=mesh2,
        scratch_types=[pltpu.VMEM((_PWB,), jnp.int32),
                       pltpu.VMEM((_PWB, _CP), f32),
                       pltpu.SemaphoreType.DMA],
    )(up.reshape(_NWB, _PWB, _CP), idxg.reshape(_NWB, _PWB))

    out = pl.pallas_call(
        _fin_body,
        grid=(_B, _NYB),
        in_specs=[pl.BlockSpec((_TB, _CP), lambda b, j: (b * _NYB + j, 0)),
                  pl.BlockSpec((_YB, _NX), lambda b, j: (b * _NYB + j, 0))],
        out_specs=pl.BlockSpec((None, _C, _YB, _NX), lambda b, j: (b, 0, j, 0)),
        out_shape=jax.ShapeDtypeStruct((_B, _C, _NY, _NX), f32),
    )(dense, mask.reshape(_B * _NY, _NX))
    return out


# speedup vs baseline: 4.1808x; 1.1050x over previous
import functools
import math

import jax
import jax.numpy as jnp
from jax import lax
from jax.experimental import pallas as pl
from jax.experimental.pallas import tpu as pltpu
from jax.experimental.pallas import tpu_sc as plsc

_NX, _NY = 216, 248
_C, _HID = 64, 64
_CP = 128
_P = 4000
_B = 2
_NCELL = _NY * _NX
_QB = 400
_YB = 8
_NYB = _NY // _YB
_TB = _YB * _NX
_NWA = 16
_PWA = (_P * _B) // _NWA
_MW = (_B * _NCELL) // _NWA
_MWPAD = ((_MW + 15) // 16) * 16
_NWB = 20
_PWB = (_P * _B) // _NWB


def _prep_body(hh_ref, ww_ref, p_ref, g1_ref, be1_ref, wq_ref, bq_ref,
               wk_ref, bk_ref, wv_ref, bv_ref,
               ne1_ref, q_ref, k_ref, v_ref):
    half = _C // 2
    j = lax.broadcasted_iota(jnp.int32, (1, half), 1).astype(jnp.float32)
    theta = jnp.exp(j * (-math.log(10000.0) / half))
    ah = hh_ref[...] * theta
    aw = ww_ref[...] * theta
    def _cos(v):
        u = v * v
        c = 1.0 / 479001600.0
        c = c * u - 1.0 / 3628800.0
        c = c * u + 1.0 / 40320.0
        c = c * u - 1.0 / 720.0
        c = c * u + 1.0 / 24.0
        c = c * u - 0.5
        return c * u + 1.0
    cd = _cos(ah - aw)
    cs = _cos(ah + aw)
    cc = 0.5 * (cd + cs)
    ss = 0.5 * (cd - cs)
    p = p_ref[...]
    x1 = p[:, :half]
    x2 = p[:, half:]
    ne1 = jnp.concatenate([x1 + x1 * cc - x2 * ss,
                           x2 + x1 * ss + x2 * cc], axis=1)
    mu = jnp.mean(ne1, axis=-1, keepdims=True)
    d = ne1 - mu
    var = jnp.mean(d * d, axis=-1, keepdims=True)
    ne = d * lax.rsqrt(var + 1e-5) * g1_ref[...] + be1_ref[...]
    ne1_ref[...] = ne1
    q_ref[...] = jnp.dot(ne, wq_ref[...], preferred_element_type=jnp.float32) + bq_ref[...]
    k_ref[...] = jnp.dot(ne, wk_ref[...], preferred_element_type=jnp.float32) + bk_ref[...]
    v = jnp.dot(ne, wv_ref[...], preferred_element_type=jnp.float32) + bv_ref[...]
    v_ref[...] = jnp.concatenate(
        [v, jnp.ones((_P, 1), jnp.float32), jnp.zeros((_P, _CP - _C - 1), jnp.float32)],
        axis=1)


def _att_body(q_ref, ne1_ref, k_ref, v_ref, g2_ref, be2_ref,
              w1_ref, b1_ref, w2_ref, b2_ref, up_ref):
    s = lax.dot_general(q_ref[...], k_ref[...], (((1,), (1,)), ((), ())),
                        preferred_element_type=jnp.float32)
    e = jnp.exp(s)
    o = jnp.dot(e, v_ref[...], preferred_element_type=jnp.float32)
    att1 = o[:, :_C] / o[:, _C:_C + 1] + ne1_ref[...]
    mu = jnp.mean(att1, axis=-1, keepdims=True)
    d = att1 - mu
    var = jnp.mean(d * d, axis=-1, keepdims=True)
    t = d * lax.rsqrt(var + 1e-5) * g2_ref[...] + be2_ref[...]
    h = jnp.dot(t, w1_ref[...], preferred_element_type=jnp.float32) + b1_ref[...]
    h = h * 0.5 * (1.0 + lax.erf(h * (1.0 / math.sqrt(2.0))))
    up = (jnp.dot(h, w2_ref[...], preferred_element_type=jnp.float32)
          + b2_ref[...] + att1)
    up_ref[...] = jnp.concatenate([up, jnp.zeros_like(up)], axis=1)


def _sc_mask_body(idxg_hbm, mask_hbm, zero_v, idx_v, ones_v, sem):
    w = lax.axis_index("s")

    zvec = jnp.zeros((16,), jnp.float32)
    def _zb(i, c):
        zero_v[pl.ds(i * 16, 16)] = zvec
        return c
    lax.fori_loop(0, _MWPAD // 16, _zb, 0)
    ovec = jnp.ones((16,), jnp.float32)
    def _ob(i, c):
        ones_v[pl.ds(i * 16, 16)] = ovec
        return c
    lax.fori_loop(0, ((_PWA + 15) // 16 * 16) // 16, _ob, 0)

    pltpu.sync_copy(zero_v.at[pl.ds(0, _MW)], mask_hbm.at[pl.ds(w * _MW, _MW)])
    plsc.subcore_barrier()

    pltpu.sync_copy(idxg_hbm.at[w], idx_v)
    pltpu.async_copy(ones_v.at[pl.ds(0, _PWA)], mask_hbm.at[idx_v], sem).wait()


def _sc_scatter_body(up_hbm, idxg_hbm, dense_hbm, idx_v, rows_v, sem):
    cid = lax.axis_index("c")
    sid = lax.axis_index("s")
    w = sid * 2 + cid

    @pl.when(w < _NWB)
    def _():
        pltpu.sync_copy(idxg_hbm.at[w], idx_v)
        pltpu.sync_copy(up_hbm.at[w], rows_v)
        pltpu.async_copy(rows_v, dense_hbm.at[idx_v], sem).wait()


def _fin_body(dense_ref, mask_ref, out_ref):
    vals = dense_ref[...]
    m = mask_ref[...]
    pieces = [vals[r * _NX:(r + 1) * _NX, :_C].T[:, None, :] for r in range(_YB)]
    t = jnp.concatenate(pieces, axis=1)
    out_ref[...] = jnp.where(m[None, :, :] > 0.5, t, 0.0)


def kernel(pillar_features, voxel_coords, batch_size, Wq, bq, Wk, bk, Wv, bv,
           W1, b1, W2, b2, g1, beta1, g2, beta2):
    f32 = jnp.float32
    p = pillar_features.astype(f32).reshape(_B, _P, _C)
    vc = voxel_coords.astype(jnp.int32)
    y = vc[:, 2].astype(f32).reshape(_B, _P, 1)
    x = vc[:, 3].astype(f32).reshape(_B, _P, 1)
    hh = y * (2.0 / (_NY - 1)) - 1.0
    ww = x * (2.0 / (_NX - 1)) - 1.0
    idxg = (vc[:, 0] * _NCELL + vc[:, 1] + vc[:, 2] * _NX + vc[:, 3])

    wspec = lambda *shape: pl.BlockSpec(shape, lambda *a: (0,) * len(shape))
    bspec = pl.BlockSpec((None, _P, _C), lambda b: (b, 0, 0))
    hspec = pl.BlockSpec((None, _P, 1), lambda b: (b, 0, 0))
    ne1, q, k, v = pl.pallas_call(
        _prep_body,
        grid=(_B,),
        in_specs=[hspec, hspec, bspec,
                  wspec(_C), wspec(_C),
                  wspec(_C, _HID), wspec(_HID),
                  wspec(_C, _HID), wspec(_HID),
                  wspec(_C, _HID), wspec(_HID)],
        out_specs=[bspec, bspec, bspec,
                   pl.BlockSpec((None, _P, _CP), lambda b: (b, 0, 0))],
        out_shape=[jax.ShapeDtypeStruct((_B, _P, _C), f32)] * 3
        + [jax.ShapeDtypeStruct((_B, _P, _CP), f32)],
    )(hh, ww, p, g1, beta1, Wq, bq, Wk, bk, Wv, bv)

    nq = _P // _QB
    qspec = pl.BlockSpec((None, _QB, _C), lambda b, i: (b, i, 0))
    up = pl.pallas_call(
        _att_body,
        grid=(_B, nq),
        in_specs=[qspec, qspec,
                  pl.BlockSpec((None, _P, _C), lambda b, i: (b, 0, 0)),
                  pl.BlockSpec((None, _P, _CP), lambda b, i: (b, 0, 0)),
                  wspec(_C), wspec(_C),
                  wspec(_HID, _HID), wspec(_HID),
                  wspec(_HID, _C), wspec(_C)],
        out_specs=pl.BlockSpec((None, _QB, _CP), lambda b, i: (b, i, 0)),
        out_shape=jax.ShapeDtypeStruct((_B, _P, _CP), f32),
    )(q, ne1, k, v, g2, beta2, W1, b1, W2, b2)

    mesh1 = plsc.VectorSubcoreMesh(core_axis_name="c", subcore_axis_name="s",
                                   num_cores=1)
    mask = pl.kernel(
        _sc_mask_body,
        out_type=jax.ShapeDtypeStruct((_B * _NCELL,), f32),
        mesh=mesh1,
        scratch_types=[pltpu.VMEM((_MWPAD,), f32),
                       pltpu.VMEM((_PWA,), jnp.int32),
                       pltpu.VMEM(((_PWA + 15) // 16 * 16,), f32),
                       pltpu.SemaphoreType.DMA],
        compiler_params=pltpu.CompilerParams(use_tc_tiling_on_sc=False),
    )(idxg.reshape(_NWA, _PWA))

    mesh2 = plsc.VectorSubcoreMesh(core_axis_name="c", subcore_axis_name="s")
    dense = pl.kernel(
        _sc_scatter_body,
        out_type=jax.ShapeDtypeStruct((_B * _NCELL, _CP), f32),
        mesh=mesh2,
        scratch_types=[pltpu.VMEM((_PWB,), jnp.int32),
                       pltpu.VMEM((_PWB, _CP), f32),
                       pltpu.SemaphoreType.DMA],
    )(up.reshape(_NWB, _PWB, _CP), idxg.reshape(_NWB, _PWB))

    out = pl.pallas_call(
        _fin_body,
        grid=(_B, _NYB),
        in_specs=[pl.BlockSpec((_TB, _CP), lambda b, j: (b * _NYB + j, 0)),
                  pl.BlockSpec((_YB, _NX), lambda b, j: (b * _NYB + j, 0))],
        out_specs=pl.BlockSpec((None, _C, _YB, _NX), lambda b, j: (b, 0, j, 0)),
        out_shape=jax.ShapeDtypeStruct((_B, _C, _NY, _NX), f32),
    )(dense, mask.reshape(_B * _NY, _NX))
    return out


# speedup vs baseline: 4.2475x; 1.0159x over previous
import functools
import math

import jax
import jax.numpy as jnp
from jax import lax
from jax.experimental import pallas as pl
from jax.experimental.pallas import tpu as pltpu
from jax.experimental.pallas import tpu_sc as plsc

_NX, _NY = 216, 248
_C, _HID = 64, 64
_CP = 128
_P = 4000
_B = 2
_NCELL = _NY * _NX
_QB = 800
_YB = 8
_NYB = _NY // _YB
_TB = _YB * _NX
_NWA = 16
_PWA = (_P * _B) // _NWA
_MW = (_B * _NCELL) // _NWA
_MWPAD = ((_MW + 15) // 16) * 16
_NWB = 20
_PWB = (_P * _B) // _NWB


def _prep_body(hh_ref, ww_ref, p_ref, g1_ref, be1_ref, wq_ref, bq_ref,
               wk_ref, bk_ref, wv_ref, bv_ref,
               ne1_ref, q_ref, k_ref, v_ref):
    half = _C // 2
    j = lax.broadcasted_iota(jnp.int32, (1, half), 1).astype(jnp.float32)
    theta = jnp.exp(j * (-math.log(10000.0) / half))
    ah = hh_ref[...] * theta
    aw = ww_ref[...] * theta
    def _cos(v):
        u = v * v
        c = 1.0 / 479001600.0
        c = c * u - 1.0 / 3628800.0
        c = c * u + 1.0 / 40320.0
        c = c * u - 1.0 / 720.0
        c = c * u + 1.0 / 24.0
        c = c * u - 0.5
        return c * u + 1.0
    cd = _cos(ah - aw)
    cs = _cos(ah + aw)
    cc = 0.5 * (cd + cs)
    ss = 0.5 * (cd - cs)
    p = p_ref[...]
    x1 = p[:, :half]
    x2 = p[:, half:]
    ne1 = jnp.concatenate([x1 + x1 * cc - x2 * ss,
                           x2 + x1 * ss + x2 * cc], axis=1)
    mu = jnp.mean(ne1, axis=-1, keepdims=True)
    d = ne1 - mu
    var = jnp.mean(d * d, axis=-1, keepdims=True)
    ne = d * lax.rsqrt(var + 1e-5) * g1_ref[...] + be1_ref[...]
    ne1_ref[...] = ne1
    q_ref[...] = jnp.dot(ne, wq_ref[...], preferred_element_type=jnp.float32) + bq_ref[...]
    k_ref[...] = jnp.dot(ne, wk_ref[...], preferred_element_type=jnp.float32) + bk_ref[...]
    v = jnp.dot(ne, wv_ref[...], preferred_element_type=jnp.float32) + bv_ref[...]
    v_ref[...] = jnp.concatenate(
        [v, jnp.ones((_P, 1), jnp.float32), jnp.zeros((_P, _CP - _C - 1), jnp.float32)],
        axis=1)


def _att_body(q_ref, ne1_ref, k_ref, v_ref, g2_ref, be2_ref,
              w1_ref, b1_ref, w2_ref, b2_ref, up_ref):
    s = lax.dot_general(q_ref[...], k_ref[...], (((1,), (1,)), ((), ())),
                        preferred_element_type=jnp.float32)
    e = jnp.exp(s)
    o = jnp.dot(e, v_ref[...], preferred_element_type=jnp.float32)
    att1 = o[:, :_C] / o[:, _C:_C + 1] + ne1_ref[...]
    mu = jnp.mean(att1, axis=-1, keepdims=True)
    d = att1 - mu
    var = jnp.mean(d * d, axis=-1, keepdims=True)
    t = d * lax.rsqrt(var + 1e-5) * g2_ref[...] + be2_ref[...]
    h = jnp.dot(t, w1_ref[...], preferred_element_type=jnp.float32) + b1_ref[...]
    h = h * 0.5 * (1.0 + lax.erf(h * (1.0 / math.sqrt(2.0))))
    up = (jnp.dot(h, w2_ref[...], preferred_element_type=jnp.float32)
          + b2_ref[...] + att1)
    up_ref[...] = jnp.concatenate([up, jnp.zeros_like(up)], axis=1)


def _sc_mask_body(idxg_hbm, mask_hbm, zero_v, idx_v, ones_v, sem):
    w = lax.axis_index("s")

    zvec = jnp.zeros((16,), jnp.float32)
    def _zb(i, c):
        zero_v[pl.ds(i * 16, 16)] = zvec
        return c
    lax.fori_loop(0, _MWPAD // 16, _zb, 0)
    ovec = jnp.ones((16,), jnp.float32)
    def _ob(i, c):
        ones_v[pl.ds(i * 16, 16)] = ovec
        return c
    lax.fori_loop(0, ((_PWA + 15) // 16 * 16) // 16, _ob, 0)

    pltpu.sync_copy(zero_v.at[pl.ds(0, _MW)], mask_hbm.at[pl.ds(w * _MW, _MW)])
    plsc.subcore_barrier()

    pltpu.sync_copy(idxg_hbm.at[w], idx_v)
    pltpu.async_copy(ones_v.at[pl.ds(0, _PWA)], mask_hbm.at[idx_v], sem).wait()


def _sc_scatter_body(up_hbm, idxg_hbm, dense_hbm, idx_v, rows_v, sem):
    cid = lax.axis_index("c")
    sid = lax.axis_index("s")
    w = sid * 2 + cid

    @pl.when(w < _NWB)
    def _():
        pltpu.sync_copy(idxg_hbm.at[w], idx_v)
        pltpu.sync_copy(up_hbm.at[w], rows_v)
        pltpu.async_copy(rows_v, dense_hbm.at[idx_v], sem).wait()


def _fin_body(dense_ref, mask_ref, out_ref):
    vals = dense_ref[...]
    m = mask_ref[...]
    pieces = [vals[r * _NX:(r + 1) * _NX, :_C].T[:, None, :] for r in range(_YB)]
    t = jnp.concatenate(pieces, axis=1)
    out_ref[...] = jnp.where(m[None, :, :] > 0.5, t, 0.0)


def kernel(pillar_features, voxel_coords, batch_size, Wq, bq, Wk, bk, Wv, bv,
           W1, b1, W2, b2, g1, beta1, g2, beta2):
    f32 = jnp.float32
    p = pillar_features.astype(f32).reshape(_B, _P, _C)
    vc = voxel_coords.astype(jnp.int32)
    y = vc[:, 2].astype(f32).reshape(_B, _P, 1)
    x = vc[:, 3].astype(f32).reshape(_B, _P, 1)
    hh = y * (2.0 / (_NY - 1)) - 1.0
    ww = x * (2.0 / (_NX - 1)) - 1.0
    idxg = (vc[:, 0] * _NCELL + vc[:, 1] + vc[:, 2] * _NX + vc[:, 3])

    wspec = lambda *shape: pl.BlockSpec(shape, lambda *a: (0,) * len(shape))
    bspec = pl.BlockSpec((None, _P, _C), lambda b: (b, 0, 0))
    hspec = pl.BlockSpec((None, _P, 1), lambda b: (b, 0, 0))
    ne1, q, k, v = pl.pallas_call(
        _prep_body,
        grid=(_B,),
        in_specs=[hspec, hspec, bspec,
                  wspec(_C), wspec(_C),
                  wspec(_C, _HID), wspec(_HID),
                  wspec(_C, _HID), wspec(_HID),
                  wspec(_C, _HID), wspec(_HID)],
        out_specs=[bspec, bspec, bspec,
                   pl.BlockSpec((None, _P, _CP), lambda b: (b, 0, 0))],
        out_shape=[jax.ShapeDtypeStruct((_B, _P, _C), f32)] * 3
        + [jax.ShapeDtypeStruct((_B, _P, _CP), f32)],
    )(hh, ww, p, g1, beta1, Wq, bq, Wk, bk, Wv, bv)

    nq = _P // _QB
    qspec = pl.BlockSpec((None, _QB, _C), lambda b, i: (b, i, 0))
    up = pl.pallas_call(
        _att_body,
        grid=(_B, nq),
        in_specs=[qspec, qspec,
                  pl.BlockSpec((None, _P, _C), lambda b, i: (b, 0, 0)),
                  pl.BlockSpec((None, _P, _CP), lambda b, i: (b, 0, 0)),
                  wspec(_C), wspec(_C),
                  wspec(_HID, _HID), wspec(_HID),
                  wspec(_HID, _C), wspec(_C)],
        out_specs=pl.BlockSpec((None, _QB, _CP), lambda b, i: (b, i, 0)),
        out_shape=jax.ShapeDtypeStruct((_B, _P, _CP), f32),
    )(q, ne1, k, v, g2, beta2, W1, b1, W2, b2)

    mesh1 = plsc.VectorSubcoreMesh(core_axis_name="c", subcore_axis_name="s",
                                   num_cores=1)
    mask = pl.kernel(
        _sc_mask_body,
        out_type=jax.ShapeDtypeStruct((_B * _NCELL,), f32),
        mesh=mesh1,
        scratch_types=[pltpu.VMEM((_MWPAD,), f32),
                       pltpu.VMEM((_PWA,), jnp.int32),
                       pltpu.VMEM(((_PWA + 15) // 16 * 16,), f32),
                       pltpu.SemaphoreType.DMA],
        compiler_params=pltpu.CompilerParams(use_tc_tiling_on_sc=False),
    )(idxg.reshape(_NWA, _PWA))

    mesh2 = plsc.VectorSubcoreMesh(core_axis_name="c", subcore_axis_name="s")
    dense = pl.kernel(
        _sc_scatter_body,
        out_type=jax.ShapeDtypeStruct((_B * _NCELL, _CP), f32),
        mesh=mesh2,
        scratch_types=[pltpu.VMEM((_PWB,), jnp.int32),
                       pltpu.VMEM((_PWB, _CP), f32),
                       pltpu.SemaphoreType.DMA],
    )(up.reshape(_NWB, _PWB, _CP), idxg.reshape(_NWB, _PWB))

    out = pl.pallas_call(
        _fin_body,
        grid=(_B, _NYB),
        in_specs=[pl.BlockSpec((_TB, _CP), lambda b, j: (b * _NYB + j, 0)),
                  pl.BlockSpec((_YB, _NX), lambda b, j: (b * _NYB + j, 0))],
        out_specs=pl.BlockSpec((None, _C, _YB, _NX), lambda b, j: (b, 0, j, 0)),
        out_shape=jax.ShapeDtypeStruct((_B, _C, _NY, _NX), f32),
    )(dense, mask.reshape(_B * _NY, _NX))
    return out
